# baseline (speedup 1.0000x reference)
import functools

import jax
import jax.numpy as jnp
from jax import lax
from jax.experimental import pallas as pl
from jax.experimental.pallas import tpu as pltpu
from jax.experimental.pallas import tpu_sc as plsc

_FEATURES = 2048
_K = 8
_XMIN = -3.0
_XMAX = 3.0
_DELTA = (_XMAX - _XMIN) / float(_K - 1)
_INV_DELTA = 1.0 / _DELTA
_POS_OFF = -_XMIN * _INV_DELTA
_POS_MAX = float(_K - 1)

_NC = 2
_NS = 16
_NW = _NC * _NS
_LANES = 16

_TAB_WORDS = _FEATURES * _K
_HI_MASK = jnp.int32(-65536)


def _sc_body(n_chunks, ch_rows, x_hbm, tab_hbm, out_hbm,
             tab_v, in0, in1, bout_v, si0, si1, so):
    wid = lax.axis_index("s") * _NC + lax.axis_index("c")
    row0 = wid * (n_chunks * ch_rows)
    pltpu.sync_copy(tab_hbm, tab_v)
    iota8 = lax.iota(jnp.int32, _LANES) * _K
    vec_per_row = _FEATURES // _LANES
    n_vec = ch_rows * vec_per_row
    ins = (in0, in1)
    sis = (si0, si1)

    def in_slice(c):
        return x_hbm.at[pl.ds(row0 + c * ch_rows, ch_rows), :]

    def out_slice(c):
        return out_hbm.at[pl.ds(row0 + c * ch_rows, ch_rows), :]

    pltpu.async_copy(in_slice(0), ins[0], sis[0])

    def pair_body(g, carry):
        for b in range(2):
            c = g * 2 + b
            pltpu.make_async_copy(in_slice(c), ins[b], sis[b]).wait()

            @pl.when(c + 1 < n_chunks)
            def _():
                pltpu.async_copy(in_slice(c + 1), ins[1 - b], sis[1 - b])

            @pl.when(c >= 1)
            def _():
                pltpu.make_async_copy(bout_v, out_slice(c - 1), so).wait()

            bin_v = ins[b]

            @plsc.parallel_loop(0, n_vec, 1, unroll=16)
            def _(i):
                r = lax.shift_right_logical(i, 7)
                jcol = (i & (vec_per_row - 1)) * _LANES
                v = bin_v[r, pl.ds(jcol, _LANES)]
                pos = jnp.minimum(
                    jnp.maximum(v * _INV_DELTA + _POS_OFF, 0.0), _POS_MAX)
                i0 = pos.astype(jnp.int32)
                frac = pos - i0.astype(jnp.float32)
                idx0 = iota8 + jcol * _K + i0
                w = plsc.load_gather(tab_v, [idx0])
                y1 = plsc.bitcast(w & _HI_MASK, jnp.float32)
                y0 = plsc.bitcast(w << 16, jnp.float32)
                bout_v[r, pl.ds(jcol, _LANES)] = y0 + frac * (y1 - y0)

            pltpu.async_copy(bout_v, out_slice(c), so)
        return carry

    lax.fori_loop(0, n_chunks // 2, pair_body, 0)
    pltpu.make_async_copy(bout_v, out_slice(n_chunks - 1), so).wait()


def _sc_spline(x2, tab, sc_rows, ch_rows=16):
    per_worker = sc_rows // _NW
    n_chunks = per_worker // ch_rows
    mesh = plsc.VectorSubcoreMesh(
        core_axis_name="c", subcore_axis_name="s",
        num_cores=_NC, num_subcores=_NS)
    fn = pl.kernel(
        functools.partial(_sc_body, n_chunks, ch_rows),
        out_type=jax.ShapeDtypeStruct((sc_rows, _FEATURES), jnp.float32),
        mesh=mesh,
        scratch_types=[
            pltpu.VMEM((_TAB_WORDS,), jnp.int32),
            pltpu.VMEM((ch_rows, _FEATURES), jnp.float32),
            pltpu.VMEM((ch_rows, _FEATURES), jnp.float32),
            pltpu.VMEM((ch_rows, _FEATURES), jnp.float32),
            pltpu.SemaphoreType.DMA,
            pltpu.SemaphoreType.DMA,
            pltpu.SemaphoreType.DMA,
        ],
        compiler_params=pltpu.CompilerParams(needs_layout_passes=False),
        cost_estimate=pl.CostEstimate(
            flops=10 * sc_rows * _FEATURES,
            bytes_accessed=8 * sc_rows * _FEATURES,
            transcendentals=0),
    )
    return fn(x2, tab)


def _pack_pair_table(knot_y):
    y0 = knot_y
    y1 = jnp.concatenate([knot_y[:, 1:], knot_y[:, -1:]], axis=1)
    lo = lax.bitcast_convert_type(
        y0.astype(jnp.bfloat16), jnp.uint16).astype(jnp.uint32)
    hi = lax.bitcast_convert_type(
        y1.astype(jnp.bfloat16), jnp.uint16).astype(jnp.uint32)
    word = (hi << 16) | lo
    return lax.bitcast_convert_type(word, jnp.int32).reshape(-1)


def _tc_spline_body(x_ref, ky_ref, o_ref):
    x = x_ref[...]
    xc = jnp.clip(x, _XMIN, _XMAX)
    pos = (xc - _XMIN) * _INV_DELTA
    idx0 = jnp.minimum(pos.astype(jnp.int32), _K - 2)
    frac = pos - idx0.astype(jnp.float32)
    y0 = ky_ref[0:1, :]
    y1 = ky_ref[1:2, :]
    for k in range(1, _K - 1):
        m = idx0 >= k
        y0 = jnp.where(m, ky_ref[k:k + 1, :], y0)
        y1 = jnp.where(m, ky_ref[k + 1:k + 2, :], y1)
    o_ref[...] = y0 + frac * (y1 - y0)


def _tc_spline_part(x2, kyT, row0, rows, br=512):
    b0 = row0 // br
    return pl.pallas_call(
        _tc_spline_body,
        grid=(rows // br,),
        in_specs=[
            pl.BlockSpec((br, _FEATURES), lambda i: (i + b0, 0)),
            pl.BlockSpec((_K, _FEATURES), lambda i: (0, 0)),
        ],
        out_specs=pl.BlockSpec((br, _FEATURES), lambda i: (i, 0)),
        out_shape=jax.ShapeDtypeStruct((rows, _FEATURES), jnp.float32),
    )(x2, kyT)


_SC_ROWS = 4096


def kernel(x, knot_y):
    rows = x.size // _FEATURES
    x2 = x.reshape(rows, _FEATURES)
    tab = _pack_pair_table(knot_y)
    sc_out = _sc_spline(x2, tab, _SC_ROWS)
    tc_out = _tc_spline_part(x2, knot_y.T, _SC_ROWS, rows - _SC_ROWS)
    return jnp.concatenate([sc_out, tc_out], axis=0).reshape(x.shape)


# speedup vs baseline: 1.1078x; 1.1078x over previous
import functools

import jax
import jax.numpy as jnp
from jax import lax
from jax.experimental import pallas as pl
from jax.experimental.pallas import tpu as pltpu
from jax.experimental.pallas import tpu_sc as plsc

_FEATURES = 2048
_K = 8
_XMIN = -3.0
_XMAX = 3.0
_DELTA = (_XMAX - _XMIN) / float(_K - 1)
_INV_DELTA = 1.0 / _DELTA
_POS_OFF = -_XMIN * _INV_DELTA
_POS_MAX = float(_K - 1)

_NC = 2
_NS = 16
_NW = _NC * _NS
_LANES = 16

_TAB_WORDS = _FEATURES * _K
_HI_MASK = jnp.int32(-65536)


def _sc_body(n_chunks, ch_rows, x_hbm, tab_hbm, out_hbm,
             tab_v, in0, in1, bout_v, si0, si1, so):
    wid = lax.axis_index("s") * _NC + lax.axis_index("c")
    row0 = wid * (n_chunks * ch_rows)
    pltpu.sync_copy(tab_hbm, tab_v)
    iota8 = lax.iota(jnp.int32, _LANES) * _K
    vec_per_row = _FEATURES // _LANES
    n_vec = ch_rows * vec_per_row
    ins = (in0, in1)
    sis = (si0, si1)

    def in_slice(c):
        return x_hbm.at[pl.ds(row0 + c * ch_rows, ch_rows), :]

    def out_slice(c):
        return out_hbm.at[pl.ds(row0 + c * ch_rows, ch_rows), :]

    pltpu.async_copy(in_slice(0), ins[0], sis[0])

    def pair_body(g, carry):
        for b in range(2):
            c = g * 2 + b
            pltpu.make_async_copy(in_slice(c), ins[b], sis[b]).wait()

            @pl.when(c + 1 < n_chunks)
            def _():
                pltpu.async_copy(in_slice(c + 1), ins[1 - b], sis[1 - b])

            @pl.when(c >= 1)
            def _():
                pltpu.make_async_copy(bout_v, out_slice(c - 1), so).wait()

            bin_v = ins[b]

            @plsc.parallel_loop(0, n_vec, 1, unroll=16)
            def _(i):
                r = lax.shift_right_logical(i, 7)
                jcol = (i & (vec_per_row - 1)) * _LANES
                v = bin_v[r, pl.ds(jcol, _LANES)]
                pos = jnp.minimum(
                    jnp.maximum(v * _INV_DELTA + _POS_OFF, 0.0), _POS_MAX)
                i0 = pos.astype(jnp.int32)
                frac = pos - i0.astype(jnp.float32)
                idx0 = iota8 + jcol * _K + i0
                w = plsc.load_gather(tab_v, [idx0])
                y1 = plsc.bitcast(w & _HI_MASK, jnp.float32)
                y0 = plsc.bitcast(w << 16, jnp.float32)
                bout_v[r, pl.ds(jcol, _LANES)] = y0 + frac * (y1 - y0)

            pltpu.async_copy(bout_v, out_slice(c), so)
        return carry

    lax.fori_loop(0, n_chunks // 2, pair_body, 0)
    pltpu.make_async_copy(bout_v, out_slice(n_chunks - 1), so).wait()


def _sc_spline(x2, tab, sc_rows, ch_rows=16):
    per_worker = sc_rows // _NW
    n_chunks = per_worker // ch_rows
    mesh = plsc.VectorSubcoreMesh(
        core_axis_name="c", subcore_axis_name="s",
        num_cores=_NC, num_subcores=_NS)
    fn = pl.kernel(
        functools.partial(_sc_body, n_chunks, ch_rows),
        out_type=jax.ShapeDtypeStruct((sc_rows, _FEATURES), jnp.float32),
        mesh=mesh,
        scratch_types=[
            pltpu.VMEM((_TAB_WORDS,), jnp.int32),
            pltpu.VMEM((ch_rows, _FEATURES), jnp.float32),
            pltpu.VMEM((ch_rows, _FEATURES), jnp.float32),
            pltpu.VMEM((ch_rows, _FEATURES), jnp.float32),
            pltpu.SemaphoreType.DMA,
            pltpu.SemaphoreType.DMA,
            pltpu.SemaphoreType.DMA,
        ],
        compiler_params=pltpu.CompilerParams(needs_layout_passes=False),
        cost_estimate=pl.CostEstimate(
            flops=10 * sc_rows * _FEATURES,
            bytes_accessed=8 * sc_rows * _FEATURES,
            transcendentals=0),
    )
    return fn(x2, tab)


def _pack_pair_table(knot_y):
    y0 = knot_y
    y1 = jnp.concatenate([knot_y[:, 1:], knot_y[:, -1:]], axis=1)
    lo = lax.bitcast_convert_type(
        y0.astype(jnp.bfloat16), jnp.uint16).astype(jnp.uint32)
    hi = lax.bitcast_convert_type(
        y1.astype(jnp.bfloat16), jnp.uint16).astype(jnp.uint32)
    word = (hi << 16) | lo
    return lax.bitcast_convert_type(word, jnp.int32).reshape(-1)


def _tc_spline_body(x_ref, ky_ref, o_ref):
    x = x_ref[...]
    xc = jnp.clip(x, _XMIN, _XMAX)
    pos = (xc - _XMIN) * _INV_DELTA
    idx0 = jnp.minimum(pos.astype(jnp.int32), _K - 2)
    frac = pos - idx0.astype(jnp.float32)
    y0 = ky_ref[0:1, :]
    y1 = ky_ref[1:2, :]
    for k in range(1, _K - 1):
        m = idx0 >= k
        y0 = jnp.where(m, ky_ref[k:k + 1, :], y0)
        y1 = jnp.where(m, ky_ref[k + 1:k + 2, :], y1)
    o_ref[...] = y0 + frac * (y1 - y0)


def _tc_spline_part(x2, kyT, row0, rows, br=512):
    b0 = row0 // br
    return pl.pallas_call(
        _tc_spline_body,
        grid=(rows // br,),
        in_specs=[
            pl.BlockSpec((br, _FEATURES), lambda i: (i + b0, 0)),
            pl.BlockSpec((_K, _FEATURES), lambda i: (0, 0)),
        ],
        out_specs=pl.BlockSpec((br, _FEATURES), lambda i: (i, 0)),
        out_shape=jax.ShapeDtypeStruct((rows, _FEATURES), jnp.float32),
    )(x2, kyT)


_SC_ROWS = 11264


def kernel(x, knot_y):
    rows = x.size // _FEATURES
    x2 = x.reshape(rows, _FEATURES)
    tab = _pack_pair_table(knot_y)
    sc_out = _sc_spline(x2, tab, _SC_ROWS)
    tc_out = _tc_spline_part(x2, knot_y.T, _SC_ROWS, rows - _SC_ROWS)
    return jnp.concatenate([sc_out, tc_out], axis=0).reshape(x.shape)
